# baseline (speedup 1.0000x reference)
import jax
import jax.numpy as jnp
from jax.experimental import pallas as pl
from jax.experimental.pallas import tpu as pltpu

_H = 128
_W = 128
_C = 80
_K = 100
_MINCONF = 0.3
_NBLK = 1024
_BLK = 1280
_NEG = -1.0
_BIG = 1 << 30


def _decode_kernel(hm_ref, off_ref, reg_ref,
                   scores_ref, classes_ref, bb_ref, v_ref):
    a = hm_ref[0]
    ninf = jnp.float32(-jnp.inf)

    wpad = jnp.full((_H, 1, _C), ninf, jnp.float32)
    left = jnp.concatenate([wpad, a[:, :-1, :]], axis=1)
    right = jnp.concatenate([a[:, 1:, :], wpad], axis=1)
    cm = jnp.maximum(a, jnp.maximum(left, right))
    hpad = jnp.full((1, _W, _C), ninf, jnp.float32)
    up = jnp.concatenate([hpad, cm[:-1, :, :]], axis=0)
    down = jnp.concatenate([cm[1:, :, :], hpad], axis=0)
    hmax = jnp.maximum(cm, jnp.maximum(up, down))
    v = jnp.where(a == hmax, a, jnp.float32(0.0))

    v5 = v.reshape(_NBLK, 16, _C)
    v_ref[...] = v5
    l1_0 = jnp.max(v5.reshape(8, 128, 16, _C), axis=(2, 3))

    iota_b = jax.lax.broadcasted_iota(jnp.int32, (8, 128), 0) * 128 + \
        jax.lax.broadcasted_iota(jnp.int32, (8, 128), 1)
    iota_o = jax.lax.broadcasted_iota(jnp.int32, (16, _C), 0) * _C + \
        jax.lax.broadcasted_iota(jnp.int32, (16, _C), 1)
    out_lane = jax.lax.broadcasted_iota(jnp.int32, (1, 128), 1)

    g0 = jnp.max(l1_0)
    b0 = jnp.min(jnp.where(l1_0 == g0, iota_b, _BIG))
    carry0 = (l1_0, g0, b0, jnp.zeros((1, 128), jnp.float32),
              jnp.zeros((1, 128), jnp.int32))

    def body(i, carry):
        l1v, g, b, s_row, f_row = carry
        blk = v_ref[pl.ds(b, 1)].reshape(16, _C)
        eq = blk == g
        o = jnp.min(jnp.where(eq, iota_o, _BIG))
        newblk = jnp.where(iota_o == o, _NEG, blk)
        v_ref[pl.ds(b, 1)] = newblk.reshape(1, 16, _C)
        nb = jnp.max(newblk)
        l1m = jnp.where(iota_b == b, jnp.float32(_NEG), l1v)
        t2 = jnp.max(l1m)
        b2 = jnp.min(jnp.where(l1m == t2, iota_b, _BIG))
        take_b = (nb > t2) | ((nb == t2) & (b < b2))
        g_n = jnp.maximum(nb, t2)
        b_n = jnp.where(take_b, b, b2)
        l1v = jnp.where(iota_b == b, nb, l1v)
        here = out_lane == i
        return (l1v, g_n, b_n,
                jnp.where(here, g, s_row),
                jnp.where(here, b * _BLK + o, f_row))

    l1v, g_l, b_l, s_row, f_row = jax.lax.fori_loop(0, _K, body, carry0)

    c = f_row % _C
    x = (f_row // _C) % _W
    y = f_row // (_W * _C)
    sub128 = jax.lax.broadcasted_iota(jnp.int32, (128, 128), 0)
    sel = (sub128 == x).astype(jnp.float32)
    dn = (((0,), (0,)), ((), ()))
    r_off = jax.lax.dot_general(off_ref[...], sel, dn,
                                preferred_element_type=jnp.float32)
    r_reg = jax.lax.dot_general(reg_ref[...], sel, dn,
                                preferred_element_type=jnp.float32)
    sub256 = jax.lax.broadcasted_iota(jnp.int32, (256, 128), 0)
    zf = jnp.float32(0.0)
    mx = sub256 == 2 * y
    my = sub256 == 2 * y + 1
    ox = jnp.sum(jnp.where(mx, r_off, zf), axis=0, keepdims=True)
    oy = jnp.sum(jnp.where(my, r_off, zf), axis=0, keepdims=True)
    rx = jnp.sum(jnp.where(mx, r_reg, zf), axis=0, keepdims=True)
    ry = jnp.sum(jnp.where(my, r_reg, zf), axis=0, keepdims=True)
    keep = s_row >= _MINCONF
    xmin = jnp.where(keep, x.astype(jnp.float32) + ox - rx * 0.5, zf)
    ymin = jnp.where(keep, y.astype(jnp.float32) + oy - ry * 0.5, zf)
    ww = jnp.where(keep, rx, zf)
    hh = jnp.where(keep, ry, zf)
    scores_ref[...] = jnp.where(keep, s_row, zf)
    classes_ref[...] = jnp.where(keep, c, 0)
    lane_m = jax.lax.broadcasted_iota(jnp.int32, (128, 128), 1)
    m_t = sub128 == lane_m
    cols = [jnp.sum(jnp.where(m_t, comp, zf), axis=1, keepdims=True)
            for comp in (xmin, ymin, ww, hh)]
    bb_ref[...] = jnp.concatenate(cols, axis=1)


@jax.jit
def kernel(heatmap, offset, regression):
    off2 = offset.reshape(_H, _W * 2)
    reg2 = regression.reshape(_H, _W * 2)
    s_out, c_out, bb_out = pl.pallas_call(
        _decode_kernel,
        out_shape=(
            jax.ShapeDtypeStruct((1, 128), jnp.float32),
            jax.ShapeDtypeStruct((1, 128), jnp.int32),
            jax.ShapeDtypeStruct((128, 4), jnp.float32),
        ),
        scratch_shapes=[
            pltpu.VMEM((_NBLK, 16, _C), jnp.float32),
        ],
    )(heatmap, off2, reg2)
    return bb_out[:_K][None], s_out[:, :_K], c_out[:, :_K]


# speedup vs baseline: 1.1836x; 1.1836x over previous
import jax
import jax.numpy as jnp
from jax.experimental import pallas as pl
from jax.experimental.pallas import tpu as pltpu

_H = 128
_W = 128
_C = 80
_K = 100
_MINCONF = 0.3
_NBLK = 1024
_BLK = 1280
_NEG = -1.0
_BIG = 1 << 30


def _decode_kernel(hm_ref, off_ref, reg_ref,
                   scores_ref, classes_ref, bb_ref, v_ref):
    a = hm_ref[0]
    ninf = jnp.float32(-jnp.inf)

    wpad = jnp.full((_H, 1, _C), ninf, jnp.float32)
    left = jnp.concatenate([wpad, a[:, :-1, :]], axis=1)
    right = jnp.concatenate([a[:, 1:, :], wpad], axis=1)
    cm = jnp.maximum(a, jnp.maximum(left, right))
    hpad = jnp.full((1, _W, _C), ninf, jnp.float32)
    up = jnp.concatenate([hpad, cm[:-1, :, :]], axis=0)
    down = jnp.concatenate([cm[1:, :, :], hpad], axis=0)
    hmax = jnp.maximum(cm, jnp.maximum(up, down))
    v = jnp.where(a == hmax, a, jnp.float32(0.0))

    v5 = v.reshape(_NBLK, 16, _C)
    v_ref[...] = v5
    l1_0 = jnp.max(v5.reshape(8, 128, 16, _C), axis=(2, 3))

    iota_b = jax.lax.broadcasted_iota(jnp.int32, (8, 128), 0) * 128 + \
        jax.lax.broadcasted_iota(jnp.int32, (8, 128), 1)
    iota_o = jax.lax.broadcasted_iota(jnp.int32, (16, _C), 0) * _C + \
        jax.lax.broadcasted_iota(jnp.int32, (16, _C), 1)
    out_lane = jax.lax.broadcasted_iota(jnp.int32, (1, 128), 1)

    g0 = jnp.max(l1_0)
    b0 = jnp.min(jnp.where(l1_0 == g0, iota_b, _BIG))
    carry0 = (l1_0, g0, b0, jnp.zeros((1, 128), jnp.float32),
              jnp.zeros((1, 128), jnp.int32))

    def body(i, carry):
        l1v, g, b, s_row, f_row = carry
        blk = v_ref[pl.ds(b, 1)].reshape(16, _C)
        eq = blk == g
        o = jnp.min(jnp.where(eq, iota_o, _BIG))
        v_ref[pl.ds(b, 1)] = jnp.where(
            iota_o == o, _NEG, blk).reshape(1, 16, _C)
        cnt = jnp.sum(jnp.where(eq, 1, 0))
        second = jnp.max(jnp.where(eq, _NEG, blk))
        nb = jnp.where(cnt > 1, g, second)
        l1m = jnp.where(iota_b == b, jnp.float32(_NEG), l1v)
        t2 = jnp.max(l1m)
        b2 = jnp.min(jnp.where(l1m == t2, iota_b, _BIG))
        take_b = (nb > t2) | ((nb == t2) & (b < b2))
        g_n = jnp.maximum(nb, t2)
        b_n = jnp.where(take_b, b, b2)
        l1v = jnp.where(iota_b == b, nb, l1v)
        here = out_lane == i
        return (l1v, g_n, b_n,
                jnp.where(here, g, s_row),
                jnp.where(here, b * _BLK + o, f_row))

    l1v, g_l, b_l, s_row, f_row = jax.lax.fori_loop(0, _K, body, carry0)

    c = f_row % _C
    x = (f_row // _C) % _W
    y = f_row // (_W * _C)
    sub128 = jax.lax.broadcasted_iota(jnp.int32, (128, 128), 0)
    sel = (sub128 == x).astype(jnp.float32)
    dn = (((0,), (0,)), ((), ()))
    r_off = jax.lax.dot_general(off_ref[...], sel, dn,
                                preferred_element_type=jnp.float32)
    r_reg = jax.lax.dot_general(reg_ref[...], sel, dn,
                                preferred_element_type=jnp.float32)
    sub256 = jax.lax.broadcasted_iota(jnp.int32, (256, 128), 0)
    zf = jnp.float32(0.0)
    mx = sub256 == 2 * y
    my = sub256 == 2 * y + 1
    ox = jnp.sum(jnp.where(mx, r_off, zf), axis=0, keepdims=True)
    oy = jnp.sum(jnp.where(my, r_off, zf), axis=0, keepdims=True)
    rx = jnp.sum(jnp.where(mx, r_reg, zf), axis=0, keepdims=True)
    ry = jnp.sum(jnp.where(my, r_reg, zf), axis=0, keepdims=True)
    keep = s_row >= _MINCONF
    xmin = jnp.where(keep, x.astype(jnp.float32) + ox - rx * 0.5, zf)
    ymin = jnp.where(keep, y.astype(jnp.float32) + oy - ry * 0.5, zf)
    ww = jnp.where(keep, rx, zf)
    hh = jnp.where(keep, ry, zf)
    scores_ref[...] = jnp.where(keep, s_row, zf)
    classes_ref[...] = jnp.where(keep, c, 0)
    lane_m = jax.lax.broadcasted_iota(jnp.int32, (128, 128), 1)
    m_t = sub128 == lane_m
    cols = [jnp.sum(jnp.where(m_t, comp, zf), axis=1, keepdims=True)
            for comp in (xmin, ymin, ww, hh)]
    bb_ref[...] = jnp.concatenate(cols, axis=1)


@jax.jit
def kernel(heatmap, offset, regression):
    off2 = offset.reshape(_H, _W * 2)
    reg2 = regression.reshape(_H, _W * 2)
    s_out, c_out, bb_out = pl.pallas_call(
        _decode_kernel,
        out_shape=(
            jax.ShapeDtypeStruct((1, 128), jnp.float32),
            jax.ShapeDtypeStruct((1, 128), jnp.int32),
            jax.ShapeDtypeStruct((128, 4), jnp.float32),
        ),
        scratch_shapes=[
            pltpu.VMEM((_NBLK, 16, _C), jnp.float32),
        ],
    )(heatmap, off2, reg2)
    return bb_out[:_K][None], s_out[:, :_K], c_out[:, :_K]
